# baseline (speedup 1.0000x reference)
import functools

import jax
import jax.numpy as jnp
from jax import lax
from jax.experimental import pallas as pl
from jax.experimental.pallas import tpu as pltpu
from jax.experimental.pallas import tpu_sc as plsc

NC, NS, LN = 2, 16, 16
NW = NC * NS
D = 1024
NV = D // LN


def _lane_sum16(x):
    i = lax.iota(jnp.int32, LN)
    for bstep in (8, 4, 2, 1):
        x = x + jnp.asarray(x).at[i ^ bstep].get(mode="promise_in_bounds")
    return x


def _newton_rsqrt_scalar(v):
    half = v * 0.5
    i = lax.bitcast_convert_type(v, jnp.int32)
    seed = jnp.int32(0x5F3759DF) - lax.shift_right_logical(i, 1)
    y = lax.bitcast_convert_type(seed, jnp.float32)
    for _ in range(4):
        y = y * (1.5 - half * y * y)
    return y


def _sc_kernel(B, L):
    R = B * L
    rows_chunk = 16
    chunk_w = rows_chunk * D
    nbuf = 4
    npf = 2
    mesh = plsc.VectorSubcoreMesh(core_axis_name="c", subcore_axis_name="s")

    @functools.partial(
        pl.kernel,
        out_type=jax.ShapeDtypeStruct((R, D), jnp.float32),
        mesh=mesh,
        scratch_types=[
            pltpu.VMEM((D,), jnp.float32),
            pltpu.VMEM((8 * D,), jnp.float32),
            pltpu.VMEM((16 * D,), jnp.float32),
            pltpu.VMEM((D,), jnp.float32),
            pltpu.VMEM((D,), jnp.float32),
            pltpu.VMEM((rows_chunk, D), jnp.float32),
            pltpu.VMEM((rows_chunk, D), jnp.float32),
            pltpu.VMEM((D,), jnp.float32),
            pltpu.VMEM((rows_chunk, D), jnp.float32),
            pltpu.VMEM((rows_chunk, D), jnp.float32),
            pltpu.VMEM((rows_chunk, D), jnp.float32),
            pltpu.VMEM((rows_chunk, D), jnp.float32),
            pltpu.SemaphoreType.DMA,
            pltpu.SemaphoreType.DMA,
            pltpu.SemaphoreType.DMA,
            pltpu.SemaphoreType.DMA,
            pltpu.SemaphoreType.DMA,
            pltpu.SemaphoreType.DMA,
            pltpu.SemaphoreType.DMA,
            pltpu.SemaphoreType.DMA,
        ],
    )
    def k(x_hbm, tt_hbm, vt_hbm, ht_hbm, w_hbm, bb_hbm, o_hbm,
          trow, vrows, hrows, wbuf, bbuf, posA, posB, tvbuf,
          r0, r1, r2, r3, si0, si1, si2, si3,
          so0, so1, so2, so3):
        c = lax.axis_index("c")
        s = lax.axis_index("s")
        t_ = s
        vbase = c * 8

        pltpu.sync_copy(tt_hbm.at[pl.ds(t_ * D, D)], trow)
        pltpu.sync_copy(vt_hbm.at[pl.ds(vbase * D, 8 * D)], vrows)
        pltpu.sync_copy(ht_hbm, hrows)
        pltpu.sync_copy(w_hbm, wbuf)
        pltpu.sync_copy(bb_hbm, bbuf)

        ring = (r0, r1, r2, r3)
        sin = (si0, si1, si2, si3)
        sout = (so0, so1, so2, so3)
        pbufs = (posA, posB)

        def chunk_off(ch):
            jj, b = divmod(ch, B)
            return b * L + t_ * 256 + (vbase + jj) * 16

        def in_copy(ch):
            return pltpu.make_async_copy(
                x_hbm.at[pl.ds(chunk_off(ch), rows_chunk)], ring[ch % nbuf],
                sin[ch % nbuf])

        def out_copy(ch):
            return pltpu.make_async_copy(
                ring[ch % nbuf], o_hbm.at[pl.ds(chunk_off(ch), rows_chunk)],
                sout[ch % nbuf])

        def compute_tv(jj):
            @plsc.parallel_loop(0, NV, unroll=8)
            def tv(kk):
                tvbuf[pl.ds(kk * LN, LN)] = (
                    trow[pl.ds(kk * LN, LN)] + vrows[pl.ds(jj * D + kk * LN, LN)])

        def compute_pos_half(h0, pbuf):
            @pl.loop(h0, h0 + rows_chunk // 2)
            def _row(h):
                zero = jnp.zeros((LN,), jnp.float32)

                @pl.loop(0, NV, init_carry=(zero, zero), unroll=8)
                def p1(kk, carry):
                    acc, acc2 = carry
                    x = (tvbuf[pl.ds(kk * LN, LN)]
                         + hrows[pl.ds(h * D + kk * LN, LN)])
                    return acc + x, acc2 + x * x

                acc, acc2 = p1
                mean_s = _lane_sum16(acc)[0] * (1.0 / D)
                ex2_s = _lane_sum16(acc2)[0] * (1.0 / D)
                var_s = ex2_s - mean_s * mean_s + 1e-6
                rs_s = _newton_rsqrt_scalar(var_s)
                mn = jnp.full((LN,), mean_s, jnp.float32)
                rs = jnp.full((LN,), rs_s, jnp.float32)

                @plsc.parallel_loop(0, NV, unroll=8)
                def p2(kk):
                    x = (tvbuf[pl.ds(kk * LN, LN)]
                         + hrows[pl.ds(h * D + kk * LN, LN)])
                    y = (x - mn) * rs
                    pbuf[h, pl.ds(pl.multiple_of(kk * LN, LN), LN)] = (
                        y * wbuf[pl.ds(kk * LN, LN)] + bbuf[pl.ds(kk * LN, LN)])

        nch = 8 * B
        for ch in range(min(npf, nch)):
            in_copy(ch).start()
        compute_tv(0)
        compute_pos_half(0, posA)
        compute_pos_half(rows_chunk // 2, posA)
        for ch in range(nch):
            p = ch % nbuf
            if ch + npf < nch:
                if ch + npf - nbuf >= 0:
                    out_copy(ch + npf - nbuf).wait()
                in_copy(ch + npf).start()
            in_copy(ch).wait()
            buf = ring[p]
            jj, b = divmod(ch, B)
            pbuf_cur = pbufs[jj % 2]

            @plsc.parallel_loop(0, chunk_w // LN, unroll=16)
            def add(kk):
                r = lax.shift_right_logical(kk, 6)
                o = pl.ds(pl.multiple_of(lax.shift_left(kk & (NV - 1), 4), LN), LN)
                plsc.addupdate(buf.at[r, o], pbuf_cur[r, o])

            out_copy(ch).start()
            if jj < 7:
                if b == 3:
                    compute_tv(jj + 1)
                    compute_pos_half(0, pbufs[(jj + 1) % 2])
                elif b == B - 1:
                    compute_pos_half(rows_chunk // 2, pbufs[(jj + 1) % 2])
        for ch in range(max(nch - nbuf, 0), nch):
            out_copy(ch).wait()
    return k


def kernel(inputs, dimensions, temporal_table, vertical_table, horizontal_table, ln_weight, ln_bias):
    B, L, Dd = inputs.shape
    flat = inputs.reshape(B * L, Dd)
    k = _sc_kernel(B, L)
    out = k(flat, temporal_table.reshape(-1), vertical_table.reshape(-1),
            horizontal_table.reshape(-1), ln_weight.reshape(-1),
            ln_bias.reshape(-1))
    return out.reshape(B, L, Dd)
